# Initial kernel scaffold; baseline (speedup 1.0000x reference)
import functools
import jax
import jax.numpy as jnp
from jax import lax
from jax.experimental import pallas as pl
from jax.experimental.pallas import tpu as pltpu
from jax.experimental.pallas import tpu_sc as plsc

N_NODES = 10000
N_EDGES = 320000
NC = 2
NS = 16
CH = 80
ROWS_PER_TILE = N_NODES // NS
ZROWS = 25
DEG_W = 16
M_BLK = 400


def _mesh():
    return plsc.VectorSubcoreMesh(core_axis_name="c", subcore_axis_name="s")


def _zero_vmem(ref, nrows, width):
    zero16 = jnp.zeros((16,), jnp.float32)

    def body(i, _):
        r = i // (width // 16)
        f = i % (width // 16)
        ref[r, pl.ds(f * 16, 16)] = zero16
        return 0

    lax.fori_loop(0, nrows * (width // 16), body, 0)


def _deg_body(ei_hbm, out_hbm, didx, ones_v, zbuf, acc):
    c = lax.axis_index("c")
    s = lax.axis_index("s")
    wid = c * NS + s
    e_per = N_EDGES // (NC * NS)
    n_chunks = e_per // CH

    one16 = jnp.ones((16,), jnp.float32)

    def fill(i, _):
        ones_v[i, :] = one16
        return 0

    lax.fori_loop(0, CH, fill, 0)
    _zero_vmem(zbuf, ZROWS, DEG_W)

    def zcopy(k, _):
        pltpu.sync_copy(zbuf, acc.at[pl.ds(s * ROWS_PER_TILE + k * ZROWS, ZROWS)])
        return 0

    lax.fori_loop(0, ROWS_PER_TILE // ZROWS, zcopy, 0)
    plsc.subcore_barrier()

    def chunk(j, _):
        off = wid * e_per + j * CH
        pltpu.sync_copy(ei_hbm.at[1, pl.ds(off, CH)], didx.at[0])
        pltpu.sync_copy(ones_v, acc.at[didx.at[0]], add=True)
        return 0

    lax.fori_loop(0, n_chunks, chunk, 0)
    plsc.subcore_barrier()

    r0 = s * ROWS_PER_TILE

    @pl.when(c == 0)
    def _():
        pltpu.sync_copy(acc.at[pl.ds(r0, ROWS_PER_TILE)],
                        out_hbm.at[0, pl.ds(r0, ROWS_PER_TILE)])

    @pl.when(c == 1)
    def _():
        pltpu.sync_copy(acc.at[pl.ds(r0, ROWS_PER_TILE)],
                        out_hbm.at[1, pl.ds(r0, ROWS_PER_TILE)])


def _sc_degree(edge_index):
    kern = pl.kernel(
        _deg_body,
        out_type=jax.ShapeDtypeStruct((NC, N_NODES, DEG_W), jnp.float32),
        mesh=_mesh(),
        scratch_types=[
            pltpu.VMEM((2, CH), jnp.int32),
            pltpu.VMEM((CH, DEG_W), jnp.float32),
            pltpu.VMEM((ZROWS, DEG_W), jnp.float32),
            pltpu.VMEM_SHARED((N_NODES, DEG_W), jnp.float32),
        ],
    )
    return kern(edge_index)


def _agg_body(H, zt_hbm, ei_hbm, out_hbm, sidx, didx, rows, zbuf, acc):
    c = lax.axis_index("c")
    s = lax.axis_index("s")
    e_per = N_EDGES // NS
    n_chunks = e_per // CH
    base = s * e_per

    _zero_vmem(zbuf, ZROWS, H)

    def zcopy(k, _):
        pltpu.sync_copy(zbuf, acc.at[pl.ds(s * ROWS_PER_TILE + k * ZROWS, ZROWS)])
        return 0

    lax.fori_loop(0, ROWS_PER_TILE // ZROWS, zcopy, 0)
    plsc.subcore_barrier()

    coff = jnp.full((16,), c * N_NODES, jnp.int32)

    def chunk(j, _):
        off = base + j * CH
        pltpu.sync_copy(ei_hbm.at[0, pl.ds(off, CH)], sidx.at[0])
        pltpu.sync_copy(ei_hbm.at[1, pl.ds(off, CH)], didx.at[0])

        def addoff(k, _):
            sidx[0, pl.ds(k * 16, 16)] = sidx[0, pl.ds(k * 16, 16)] + coff
            return 0

        lax.fori_loop(0, CH // 16, addoff, 0)
        pltpu.sync_copy(zt_hbm.at[sidx.at[0]], rows)
        pltpu.sync_copy(rows, acc.at[didx.at[0]], add=True)
        return 0

    lax.fori_loop(0, n_chunks, chunk, 0)
    plsc.subcore_barrier()

    r0 = s * ROWS_PER_TILE

    @pl.when(c == 0)
    def _():
        pltpu.sync_copy(acc.at[pl.ds(r0, ROWS_PER_TILE)],
                        out_hbm.at[0, pl.ds(r0, ROWS_PER_TILE)])

    @pl.when(c == 1)
    def _():
        pltpu.sync_copy(acc.at[pl.ds(r0, ROWS_PER_TILE)],
                        out_hbm.at[1, pl.ds(r0, ROWS_PER_TILE)])


def _sc_aggregate(zt_flat, edge_index, H):
    kern = pl.kernel(
        functools.partial(_agg_body, H),
        out_type=jax.ShapeDtypeStruct((NC, N_NODES, H), jnp.float32),
        mesh=_mesh(),
        scratch_types=[
            pltpu.VMEM((2, CH), jnp.int32),
            pltpu.VMEM((2, CH), jnp.int32),
            pltpu.VMEM((CH, H), jnp.float32),
            pltpu.VMEM((ZROWS, H), jnp.float32),
            pltpu.VMEM_SHARED((N_NODES, H), jnp.float32),
        ],
    )
    return kern(zt_flat, edge_index)


def _dinv_from(deg_ref):
    deg = deg_ref[0, :, 0] + deg_ref[1, :, 0] + 1.0
    return lax.rsqrt(deg)[:, None]


def _tc1_body(x_ref, w_ref, deg_ref, o_ref):
    z = jnp.dot(x_ref[...], w_ref[...], preferred_element_type=jnp.float32)
    zt = z * _dinv_from(deg_ref)
    h = zt.shape[1] // 2
    o_ref[0] = zt[:, :h]
    o_ref[1] = zt[:, h:]


def _tc_layer1(x, W1, deg2):
    K = x.shape[1]
    D = W1.shape[1]
    grid = N_NODES // M_BLK
    return pl.pallas_call(
        _tc1_body,
        grid=(grid,),
        in_specs=[
            pl.BlockSpec((M_BLK, K), lambda m: (m, 0)),
            pl.BlockSpec((K, D), lambda m: (0, 0)),
            pl.BlockSpec((2, M_BLK, DEG_W), lambda m: (0, m, 0)),
        ],
        out_specs=pl.BlockSpec((2, M_BLK, D // 2), lambda m: (0, m, 0)),
        out_shape=jax.ShapeDtypeStruct((2, N_NODES, D // 2), jnp.float32),
    )(x, W1, deg2)


def _tc_mid_body(relu, agg_ref, zt_ref, deg_ref, b_ref, w_ref, o_ref):
    dinv = _dinv_from(deg_ref)
    agg = jnp.concatenate([agg_ref[0], agg_ref[1]], axis=1)
    zt = jnp.concatenate([zt_ref[0], zt_ref[1]], axis=1)
    h = dinv * (agg + zt) + b_ref[...][None, :]
    if relu:
        h = jnp.maximum(h, 0.0)
    z2 = jnp.dot(h, w_ref[...], preferred_element_type=jnp.float32)
    zt2 = z2 * dinv
    hw = zt2.shape[1] // 2
    o_ref[0] = zt2[:, :hw]
    o_ref[1] = zt2[:, hw:]


def _tc_mid(agg, zt, deg2, b, W, relu):
    D = b.shape[0]
    D2 = W.shape[1]
    grid = N_NODES // M_BLK
    return pl.pallas_call(
        functools.partial(_tc_mid_body, relu),
        grid=(grid,),
        in_specs=[
            pl.BlockSpec((2, M_BLK, D // 2), lambda m: (0, m, 0)),
            pl.BlockSpec((2, M_BLK, D // 2), lambda m: (0, m, 0)),
            pl.BlockSpec((2, M_BLK, DEG_W), lambda m: (0, m, 0)),
            pl.BlockSpec((D,), lambda m: (0,)),
            pl.BlockSpec((D, D2), lambda m: (0, 0)),
        ],
        out_specs=pl.BlockSpec((2, M_BLK, D2 // 2), lambda m: (0, m, 0)),
        out_shape=jax.ShapeDtypeStruct((2, N_NODES, D2 // 2), jnp.float32),
    )(agg, zt, deg2, b, W)


def _tc_final_body(agg_ref, zt_ref, deg_ref, b_ref, wl_ref, bl_ref, h_ref, o_ref):
    dinv = _dinv_from(deg_ref)
    agg = jnp.concatenate([agg_ref[0], agg_ref[1]], axis=1)
    zt = jnp.concatenate([zt_ref[0], zt_ref[1]], axis=1)
    h3 = dinv * (agg + zt) + b_ref[...][None, :]
    h_ref[...] = h3
    logits = jnp.dot(h3, wl_ref[...], preferred_element_type=jnp.float32)
    o_ref[...] = jax.nn.sigmoid(logits + bl_ref[0])


def _tc_final(agg, zt, deg2, b3, Wl, bl):
    D = b3.shape[0]
    grid = N_NODES // M_BLK
    return pl.pallas_call(
        _tc_final_body,
        grid=(grid,),
        in_specs=[
            pl.BlockSpec((2, M_BLK, D // 2), lambda m: (0, m, 0)),
            pl.BlockSpec((2, M_BLK, D // 2), lambda m: (0, m, 0)),
            pl.BlockSpec((2, M_BLK, DEG_W), lambda m: (0, m, 0)),
            pl.BlockSpec((D,), lambda m: (0,)),
            pl.BlockSpec((D, 1), lambda m: (0, 0)),
            pl.BlockSpec((1,), lambda m: (0,)),
        ],
        out_specs=[
            pl.BlockSpec((M_BLK, D), lambda m: (m, 0)),
            pl.BlockSpec((M_BLK, 1), lambda m: (m, 0)),
        ],
        out_shape=[
            jax.ShapeDtypeStruct((N_NODES, D), jnp.float32),
            jax.ShapeDtypeStruct((N_NODES, 1), jnp.float32),
        ],
    )(agg, zt, deg2, b3, Wl, bl)


def kernel(x, W1, b1, W2, b2, W3, b3, Wl, bl, edge_index):
    deg2 = _sc_degree(edge_index)

    zt1 = _tc_layer1(x, W1, deg2)
    agg1 = _sc_aggregate(zt1.reshape(2 * N_NODES, 128), edge_index, 128)

    zt2 = _tc_mid(agg1, zt1, deg2, b1, W2, relu=True)
    agg2 = _sc_aggregate(zt2.reshape(2 * N_NODES, 64), edge_index, 64)

    zt3 = _tc_mid(agg2, zt2, deg2, b2, W3, relu=False)
    agg3 = _sc_aggregate(zt3.reshape(2 * N_NODES, 32), edge_index, 32)

    h3, out = _tc_final(agg3, zt3, deg2, b3, Wl, bl)
    return (out, h3)


# speedup vs baseline: 8.6766x; 8.6766x over previous
import functools
import jax
import jax.numpy as jnp
from jax import lax
from jax.experimental import pallas as pl
from jax.experimental.pallas import tpu as pltpu
from jax.experimental.pallas import tpu_sc as plsc

N_NODES = 10000
N_PAD = 10240
N_EDGES = 320000
NC = 2
NS = 16
CH = 80
ROWS_PER_TILE = N_PAD // NS
ZROWS = 32
DEG_W = 128
M_BLK = 400


def _mesh():
    return plsc.VectorSubcoreMesh(core_axis_name="c", subcore_axis_name="s")


def _zero_vmem(ref, nrows, width):
    zero16 = jnp.zeros((16,), jnp.float32)

    def body(i, _):
        r = i // (width // 16)
        f = i % (width // 16)
        ref[r, pl.ds(f * 16, 16)] = zero16
        return 0

    lax.fori_loop(0, nrows * (width // 16), body, 0)


def _deg_body(dst_hbm, out_hbm, didx, ones_v, zbuf, acc):
    c = lax.axis_index("c")
    s = lax.axis_index("s")
    wid = c * NS + s
    e_per = N_EDGES // (NC * NS)
    n_chunks = e_per // CH

    one16 = jnp.ones((16,), jnp.float32)

    def fill(i, _):
        r = i // (DEG_W // 16)
        f = i % (DEG_W // 16)
        ones_v[r, pl.ds(f * 16, 16)] = one16
        return 0

    lax.fori_loop(0, CH * (DEG_W // 16), fill, 0)
    _zero_vmem(zbuf, ZROWS, DEG_W)

    def zcopy(k, _):
        pltpu.sync_copy(zbuf, acc.at[pl.ds(s * ROWS_PER_TILE + k * ZROWS, ZROWS)])
        return 0

    lax.fori_loop(0, ROWS_PER_TILE // ZROWS, zcopy, 0)
    plsc.subcore_barrier()

    def chunk(j, _):
        off = wid * e_per + j * CH
        pltpu.sync_copy(dst_hbm.at[pl.ds(off, CH)], didx.at[0])
        pltpu.sync_copy(ones_v, acc.at[didx.at[0]], add=True)
        return 0

    lax.fori_loop(0, n_chunks, chunk, 0)
    plsc.subcore_barrier()

    r0 = s * ROWS_PER_TILE

    @pl.when(c == 0)
    def _():
        pltpu.sync_copy(acc.at[pl.ds(r0, ROWS_PER_TILE)],
                        out_hbm.at[0, pl.ds(r0, ROWS_PER_TILE)])

    @pl.when(c == 1)
    def _():
        pltpu.sync_copy(acc.at[pl.ds(r0, ROWS_PER_TILE)],
                        out_hbm.at[1, pl.ds(r0, ROWS_PER_TILE)])


def _sc_degree(dst):
    kern = pl.kernel(
        _deg_body,
        out_type=jax.ShapeDtypeStruct((NC, N_PAD, DEG_W), jnp.float32),
        mesh=_mesh(),
        scratch_types=[
            pltpu.VMEM((2, CH), jnp.int32),
            pltpu.VMEM((CH, DEG_W), jnp.float32),
            pltpu.VMEM((ZROWS, DEG_W), jnp.float32),
            pltpu.VMEM_SHARED((N_PAD, DEG_W), jnp.float32),
        ],
    )
    return kern(dst)


def _agg_body(H, esplit, zt_hbm, src_hbm, dst_hbm, out_hbm, sidx, didx, rows, zbuf, acc):
    c = lax.axis_index("c")
    s = lax.axis_index("s")
    if esplit:
        e_per = N_EDGES // (NC * NS)
        wid = c * NS + s
        base = wid * e_per
    else:
        e_per = N_EDGES // NS
        base = s * e_per
    n_chunks = e_per // CH

    _zero_vmem(zbuf, ZROWS, H)

    def zcopy(k, _):
        pltpu.sync_copy(zbuf, acc.at[pl.ds(s * ROWS_PER_TILE + k * ZROWS, ZROWS)])
        return 0

    lax.fori_loop(0, ROWS_PER_TILE // ZROWS, zcopy, 0)
    plsc.subcore_barrier()

    coff = jnp.full((16,), c * N_NODES, jnp.int32)

    def chunk(j, _):
        off = base + j * CH
        pltpu.sync_copy(src_hbm.at[pl.ds(off, CH)], sidx.at[0])
        pltpu.sync_copy(dst_hbm.at[pl.ds(off, CH)], didx.at[0])

        if not esplit:
            def addoff(k, _):
                sidx[0, pl.ds(k * 16, 16)] = sidx[0, pl.ds(k * 16, 16)] + coff
                return 0

            lax.fori_loop(0, CH // 16, addoff, 0)
        pltpu.sync_copy(zt_hbm.at[sidx.at[0]], rows)
        pltpu.sync_copy(rows, acc.at[didx.at[0]], add=True)
        return 0

    lax.fori_loop(0, n_chunks, chunk, 0)
    plsc.subcore_barrier()

    r0 = s * ROWS_PER_TILE

    @pl.when(c == 0)
    def _():
        pltpu.sync_copy(acc.at[pl.ds(r0, ROWS_PER_TILE)],
                        out_hbm.at[0, pl.ds(r0, ROWS_PER_TILE)])

    @pl.when(c == 1)
    def _():
        pltpu.sync_copy(acc.at[pl.ds(r0, ROWS_PER_TILE)],
                        out_hbm.at[1, pl.ds(r0, ROWS_PER_TILE)])


def _sc_aggregate(zt_flat, src, dst, H, esplit):
    kern = pl.kernel(
        functools.partial(_agg_body, H, esplit),
        out_type=jax.ShapeDtypeStruct((NC, N_PAD, H), jnp.float32),
        mesh=_mesh(),
        scratch_types=[
            pltpu.VMEM((2, CH), jnp.int32),
            pltpu.VMEM((2, CH), jnp.int32),
            pltpu.VMEM((CH, H), jnp.float32),
            pltpu.VMEM((ZROWS, H), jnp.float32),
            pltpu.VMEM_SHARED((N_PAD, H), jnp.float32),
        ],
    )
    return kern(zt_flat, src, dst)


def _dinv_from(deg_ref):
    deg = deg_ref[0, :, 0] + deg_ref[1, :, 0] + 1.0
    return lax.rsqrt(deg)[:, None]


def _tc1_body(x_ref, w_ref, deg_ref, o_ref):
    z = jnp.dot(x_ref[...], w_ref[...], preferred_element_type=jnp.float32)
    zt = z * _dinv_from(deg_ref)
    h = zt.shape[1] // 2
    o_ref[0] = zt[:, :h]
    o_ref[1] = zt[:, h:]


def _tc_layer1(x, W1, deg2):
    K = x.shape[1]
    D = W1.shape[1]
    grid = N_NODES // M_BLK
    return pl.pallas_call(
        _tc1_body,
        grid=(grid,),
        in_specs=[
            pl.BlockSpec((M_BLK, K), lambda m: (m, 0)),
            pl.BlockSpec((K, D), lambda m: (0, 0)),
            pl.BlockSpec((2, M_BLK, DEG_W), lambda m: (0, m, 0)),
        ],
        out_specs=pl.BlockSpec((2, M_BLK, D // 2), lambda m: (0, m, 0)),
        out_shape=jax.ShapeDtypeStruct((2, N_NODES, D // 2), jnp.float32),
    )(x, W1, deg2)


def _tc_mid_body(relu, D, D2, split_in, agg_ref, zt_ref, deg_ref, b_ref, w_ref, o_ref):
    dinv = _dinv_from(deg_ref)
    if split_in:
        agg = jnp.concatenate([agg_ref[0], agg_ref[1]], axis=1)
        zt = jnp.concatenate([zt_ref[0], zt_ref[1]], axis=1)
    else:
        agg = (agg_ref[0] + agg_ref[1])[:, :D]
        zt = zt_ref[...][:, :D]
    h = dinv * (agg + zt) + b_ref[...][None, :]
    if relu:
        h = jnp.maximum(h, 0.0)
    z2 = jnp.dot(h, w_ref[...], preferred_element_type=jnp.float32)
    zt2 = z2 * dinv
    if D2 < 128:
        zt2 = jnp.concatenate(
            [zt2, jnp.zeros((zt2.shape[0], 128 - D2), jnp.float32)], axis=1)
    o_ref[...] = zt2


def _tc_mid(agg, zt, deg2, b, W, relu, split_in):
    D = b.shape[0]
    D2 = W.shape[1]
    grid = N_NODES // M_BLK
    if split_in:
        zt_spec = pl.BlockSpec((2, M_BLK, D // 2), lambda m: (0, m, 0))
    else:
        zt_spec = pl.BlockSpec((M_BLK, 128), lambda m: (m, 0))
    return pl.pallas_call(
        functools.partial(_tc_mid_body, relu, D, D2, split_in),
        grid=(grid,),
        in_specs=[
            pl.BlockSpec((2, M_BLK, 128), lambda m: (0, m, 0)),
            zt_spec,
            pl.BlockSpec((2, M_BLK, DEG_W), lambda m: (0, m, 0)),
            pl.BlockSpec((D,), lambda m: (0,)),
            pl.BlockSpec((D, D2), lambda m: (0, 0)),
        ],
        out_specs=pl.BlockSpec((M_BLK, 128), lambda m: (m, 0)),
        out_shape=jax.ShapeDtypeStruct((N_NODES, 128), jnp.float32),
    )(agg, zt, deg2, b, W)


def _tc_final_body(agg_ref, zt_ref, deg_ref, b_ref, wl_ref, bl_ref, h_ref, o_ref):
    D = b_ref.shape[0]
    dinv = _dinv_from(deg_ref)
    agg = (agg_ref[0] + agg_ref[1])[:, :D]
    zt = zt_ref[...][:, :D]
    h3 = dinv * (agg + zt) + b_ref[...][None, :]
    h_ref[...] = h3
    logits = jnp.dot(h3, wl_ref[...], preferred_element_type=jnp.float32)
    o_ref[...] = jax.nn.sigmoid(logits + bl_ref[0])


def _tc_final(agg, zt, deg2, b3, Wl, bl):
    D = b3.shape[0]
    grid = N_NODES // M_BLK
    return pl.pallas_call(
        _tc_final_body,
        grid=(grid,),
        in_specs=[
            pl.BlockSpec((2, M_BLK, 128), lambda m: (0, m, 0)),
            pl.BlockSpec((M_BLK, 128), lambda m: (m, 0)),
            pl.BlockSpec((2, M_BLK, DEG_W), lambda m: (0, m, 0)),
            pl.BlockSpec((D,), lambda m: (0,)),
            pl.BlockSpec((D, 1), lambda m: (0, 0)),
            pl.BlockSpec((1,), lambda m: (0,)),
        ],
        out_specs=[
            pl.BlockSpec((M_BLK, D), lambda m: (m, 0)),
            pl.BlockSpec((M_BLK, 1), lambda m: (m, 0)),
        ],
        out_shape=[
            jax.ShapeDtypeStruct((N_NODES, D), jnp.float32),
            jax.ShapeDtypeStruct((N_NODES, 1), jnp.float32),
        ],
    )(agg, zt, deg2, b3, Wl, bl)


def kernel(x, W1, b1, W2, b2, W3, b3, Wl, bl, edge_index):
    src = edge_index[0]
    dst = edge_index[1]
    deg2 = _sc_degree(dst)

    zt1 = _tc_layer1(x, W1, deg2)
    agg1 = _sc_aggregate(zt1.reshape(2 * N_NODES, 128), src, dst, 128,
                         esplit=False)

    zt2 = _tc_mid(agg1, zt1, deg2, b1, W2, relu=True,
                  split_in=True)
    agg2 = _sc_aggregate(zt2, src, dst, 128, esplit=True)

    zt3 = _tc_mid(agg2, zt2, deg2, b2, W3, relu=False,
                  split_in=False)
    agg3 = _sc_aggregate(zt3, src, dst, 128, esplit=True)

    h3, out = _tc_final(agg3, zt3, deg2, b3, Wl, bl)
    return (out, h3)
